# Initial kernel scaffold; baseline (speedup 1.0000x reference)
import jax
import jax.numpy as jnp
from jax.experimental import pallas as pl


def kernel(obs_vec, zone_var_index, zone_mask, edge_index, W_enc, b_enc, W_gcn, b_gcn, W_q1, b_q1, W_q2, b_q2):
    raise NotImplementedError("write your pallas kernel here")


# speedup vs baseline: 8.1464x; 8.1464x over previous
import functools
import jax
import jax.numpy as jnp
from jax.experimental import pallas as pl
from jax.experimental.pallas import tpu as pltpu
from jax.experimental.pallas import tpu_sc as plsc

N = 10000
F = 128
E = 320000
OBS = N * F
A = 10

NC = 2
NS = 16
NW = NC * NS

NP = 10240
RPT = NP // NW
EP = 327680
GPT = EP // (NW * 128)
DPAD = NP


def _sc1_body(obs_hbm, zvi_hbm, dst_hbm, ones_hbm, zeros1_hbm,
              feat_hbm, deg_hbm, idx_v, val_v, dst_v, ones_v, deg_sp):
  c = jax.lax.axis_index("c")
  s = jax.lax.axis_index("s")
  wid = s * NC + c

  @pl.when(s == 0)
  def _():
    pltpu.sync_copy(zeros1_hbm, deg_sp)
  plsc.subcore_barrier()

  pltpu.sync_copy(ones_hbm, ones_v)
  pltpu.sync_copy(dst_hbm.at[pl.ds(wid * GPT, GPT)], dst_v)

  @pl.loop(0, GPT)
  def _(j):
    pltpu.sync_copy(ones_v.at[j], deg_sp.at[dst_v.at[j]], add=True)

  pltpu.sync_copy(zvi_hbm.at[pl.ds(wid * RPT, RPT)], idx_v)

  @pl.loop(0, RPT)
  def _(j):
    pltpu.sync_copy(obs_hbm.at[idx_v.at[j]], val_v.at[j])

  pltpu.sync_copy(val_v, feat_hbm.at[pl.ds(wid * RPT, RPT)])

  plsc.subcore_barrier()
  pltpu.sync_copy(deg_sp.at[pl.ds(s * (DPAD // NS), DPAD // NS)],
                  deg_hbm.at[c].at[pl.ds(s * (DPAD // NS), DPAD // NS)])


def _sc1(obs_vec, zvi_pad, dst_pad, ones128, zeros1):
  mesh = plsc.VectorSubcoreMesh(core_axis_name="c", subcore_axis_name="s")
  f = pl.kernel(
      _sc1_body,
      out_type=(jax.ShapeDtypeStruct((NP, 128), jnp.float32),
                jax.ShapeDtypeStruct((NC, DPAD), jnp.float32)),
      mesh=mesh,
      scratch_types=[
          pltpu.VMEM((RPT, 128), jnp.int32),
          pltpu.VMEM((RPT, 128), jnp.float32),
          pltpu.VMEM((GPT, 128), jnp.int32),
          pltpu.VMEM((GPT, 128), jnp.float32),
          pltpu.VMEM_SHARED((DPAD,), jnp.float32),
      ],
  )
  return f(obs_vec, zvi_pad, dst_pad, ones128, zeros1)


def _sc2_body(y_hbm, src_hbm, dst_hbm, zeros2_hbm, s_hbm,
              src_v, dst_v, rows_v, s_sp):
  c = jax.lax.axis_index("c")
  s = jax.lax.axis_index("s")
  wid = s * NC + c

  pltpu.sync_copy(zeros2_hbm.at[pl.ds(s * (DPAD // NS), DPAD // NS)],
                  s_sp.at[pl.ds(s * (DPAD // NS), DPAD // NS)])
  plsc.subcore_barrier()

  pltpu.sync_copy(src_hbm.at[pl.ds(wid * GPT, GPT)], src_v)
  pltpu.sync_copy(dst_hbm.at[pl.ds(wid * GPT, GPT)], dst_v)

  @pl.loop(0, GPT)
  def _(j):
    pltpu.sync_copy(y_hbm.at[src_v.at[j]], rows_v)
    pltpu.sync_copy(rows_v, s_sp.at[dst_v.at[j]], add=True)

  plsc.subcore_barrier()
  pltpu.sync_copy(s_sp.at[pl.ds(s * (DPAD // NS), DPAD // NS)],
                  s_hbm.at[c].at[pl.ds(s * (DPAD // NS), DPAD // NS)])


def _sc2(y, src_pad, dst_pad, zeros2):
  mesh = plsc.VectorSubcoreMesh(core_axis_name="c", subcore_axis_name="s")
  f = pl.kernel(
      _sc2_body,
      out_type=jax.ShapeDtypeStruct((NC, DPAD, 128), jnp.float32),
      mesh=mesh,
      scratch_types=[
          pltpu.VMEM((GPT, 128), jnp.int32),
          pltpu.VMEM((GPT, 128), jnp.int32),
          pltpu.VMEM((128, 128), jnp.float32),
          pltpu.VMEM_SHARED((DPAD, 128), jnp.float32),
      ],
  )
  return f(y, src_pad, dst_pad, zeros2)


BN = 1000


def _tc1_body(f_ref, m_ref, d0_ref, d1_ref, we_ref, be_ref, wg_ref, y_ref):
  x = f_ref[...] * m_ref[...]
  h1 = jnp.maximum(
      jnp.dot(x, we_ref[...], preferred_element_type=jnp.float32) + be_ref[...],
      0.0)
  dinv = jax.lax.rsqrt(1.0 + d0_ref[...] + d1_ref[...])
  y_ref[...] = dinv * jnp.dot(h1, wg_ref[...],
                              preferred_element_type=jnp.float32)


def _tc1(feature, mask, d0, d1, W_enc, b_enc, W_gcn):
  grid = (N // BN,)
  return pl.pallas_call(
      _tc1_body,
      grid=grid,
      in_specs=[
          pl.BlockSpec((BN, 128), lambda i: (i, 0)),
          pl.BlockSpec((BN, 128), lambda i: (i, 0)),
          pl.BlockSpec((BN, 1), lambda i: (i, 0)),
          pl.BlockSpec((BN, 1), lambda i: (i, 0)),
          pl.BlockSpec((128, 128), lambda i: (0, 0)),
          pl.BlockSpec((1, 128), lambda i: (0, 0)),
          pl.BlockSpec((128, 128), lambda i: (0, 0)),
      ],
      out_specs=pl.BlockSpec((BN, 128), lambda i: (i, 0)),
      out_shape=jax.ShapeDtypeStruct((N, 128), jnp.float32),
  )(feature, mask, d0, d1, W_enc, b_enc, W_gcn)


def _tc2_body(s0_ref, s1_ref, y_ref, d0_ref, d1_ref, bg_ref,
              w1_ref, b1_ref, w2_ref, b2_ref, q_ref):
  dinv = jax.lax.rsqrt(1.0 + d0_ref[...] + d1_ref[...])
  agg = dinv * (s0_ref[...] + s1_ref[...] + y_ref[...])
  h2 = jnp.maximum(agg + bg_ref[...], 0.0)
  t = jnp.maximum(
      jnp.dot(h2, w1_ref[...], preferred_element_type=jnp.float32) + b1_ref[...],
      0.0)
  q_ref[...] = jnp.dot(t, w2_ref[...],
                       preferred_element_type=jnp.float32) + b2_ref[...]


def _tc2(s0, s1, y, d0, d1, b_gcn, W_q1, b_q1, W_q2p, b_q2p):
  grid = (N // BN,)
  return pl.pallas_call(
      _tc2_body,
      grid=grid,
      in_specs=[
          pl.BlockSpec((BN, 128), lambda i: (i, 0)),
          pl.BlockSpec((BN, 128), lambda i: (i, 0)),
          pl.BlockSpec((BN, 128), lambda i: (i, 0)),
          pl.BlockSpec((BN, 1), lambda i: (i, 0)),
          pl.BlockSpec((BN, 1), lambda i: (i, 0)),
          pl.BlockSpec((1, 128), lambda i: (0, 0)),
          pl.BlockSpec((128, 128), lambda i: (0, 0)),
          pl.BlockSpec((1, 128), lambda i: (0, 0)),
          pl.BlockSpec((128, 128), lambda i: (0, 0)),
          pl.BlockSpec((1, 128), lambda i: (0, 0)),
      ],
      out_specs=pl.BlockSpec((BN, 128), lambda i: (i, 0)),
      out_shape=jax.ShapeDtypeStruct((N, 128), jnp.float32),
  )(s0, s1, y, d0, d1, b_gcn, W_q1, b_q1, W_q2p, b_q2p)


@jax.jit
def kernel(obs_vec, zone_var_index, zone_mask, edge_index, W_enc, b_enc,
           W_gcn, b_gcn, W_q1, b_q1, W_q2, b_q2):
  zvi = zone_var_index.astype(jnp.int32)
  zvi_pad = jnp.concatenate([zvi, jnp.zeros((NP - N, F), jnp.int32)], axis=0)
  src = edge_index[0].astype(jnp.int32)
  dst = edge_index[1].astype(jnp.int32)
  src_pad = jnp.concatenate([src, jnp.zeros((EP - E,), jnp.int32)])
  dst_pad = jnp.concatenate([dst, jnp.full((EP - E,), N, jnp.int32)])
  src2d = src_pad.reshape(EP // 128, 128)
  dst2d = dst_pad.reshape(EP // 128, 128)
  ones128 = jnp.ones((GPT, 128), jnp.float32)
  zeros1 = jnp.zeros((DPAD,), jnp.float32)
  zeros2 = jnp.zeros((DPAD, 128), jnp.float32)

  feat_pad, deg = _sc1(obs_vec, zvi_pad, dst2d, ones128, zeros1)
  feature = feat_pad[:N]
  d0 = deg[0, :N].reshape(N, 1)
  d1 = deg[1, :N].reshape(N, 1)

  y = _tc1(feature, zone_mask, d0, d1, W_enc, b_enc.reshape(1, 128), W_gcn)

  s_part = _sc2(y, src2d, dst2d, zeros2)

  W_q2p = jnp.pad(W_q2, ((0, 0), (0, 128 - A)))
  b_q2p = jnp.pad(b_q2, (0, 128 - A)).reshape(1, 128)
  qp = _tc2(s_part[0, :N], s_part[1, :N], y, d0, d1, b_gcn.reshape(1, 128),
            W_q1, b_q1.reshape(1, 128), W_q2p, b_q2p)
  return qp[:, :A]


# speedup vs baseline: 9.4665x; 1.1620x over previous
import functools
import jax
import jax.numpy as jnp
from jax.experimental import pallas as pl
from jax.experimental.pallas import tpu as pltpu
from jax.experimental.pallas import tpu_sc as plsc

N = 10000
F = 128
E = 320000
OBS = N * F
A = 10

NC = 2
NS = 16
NW = NC * NS

NP = 10240
RPT = NP // NW
EP = 327680
GPT = EP // (NW * 128)
G64 = EP // (NW * 64)
DPAD = NP


KAHEAD = 16


def _sc1_body(obs_hbm, zvi_hbm, dst_hbm, ones_hbm, zeros1_hbm,
              feat_hbm, deg_hbm, idx_v, val_v, dst_v, ones_v, sem_g, deg_sp):
  c = jax.lax.axis_index("c")
  s = jax.lax.axis_index("s")
  wid = s * NC + c

  @pl.when(s == 0)
  def _():
    pltpu.sync_copy(zeros1_hbm, deg_sp)
  plsc.subcore_barrier()

  pltpu.sync_copy(ones_hbm, ones_v)
  pltpu.sync_copy(dst_hbm.at[pl.ds(wid * GPT, GPT)], dst_v)

  @pl.loop(0, GPT)
  def _(j):
    pltpu.sync_copy(ones_v, deg_sp.at[dst_v.at[j]], add=True)

  pltpu.sync_copy(zvi_hbm.at[pl.ds(wid * RPT, RPT)], idx_v)

  @pl.loop(0, KAHEAD)
  def _(j):
    pltpu.async_copy(obs_hbm.at[idx_v.at[j]], val_v.at[j], sem_g)

  @pl.loop(0, RPT - KAHEAD)
  def _(j):
    pltpu.make_async_copy(obs_hbm.at[idx_v.at[j]], val_v.at[j], sem_g).wait()
    pltpu.async_copy(obs_hbm.at[idx_v.at[j + KAHEAD]], val_v.at[j + KAHEAD],
                     sem_g)

  @pl.loop(RPT - KAHEAD, RPT)
  def _(j):
    pltpu.make_async_copy(obs_hbm.at[idx_v.at[j]], val_v.at[j], sem_g).wait()

  pltpu.sync_copy(val_v, feat_hbm.at[pl.ds(wid * RPT, RPT)])

  plsc.subcore_barrier()
  pltpu.sync_copy(deg_sp.at[pl.ds(s * (DPAD // NS), DPAD // NS)],
                  deg_hbm.at[c].at[pl.ds(s * (DPAD // NS), DPAD // NS)])


def _sc1(obs_vec, zvi_pad, dst_pad, ones128, zeros1):
  mesh = plsc.VectorSubcoreMesh(core_axis_name="c", subcore_axis_name="s")
  f = pl.kernel(
      _sc1_body,
      out_type=(jax.ShapeDtypeStruct((NP, 128), jnp.float32),
                jax.ShapeDtypeStruct((NC, DPAD), jnp.float32)),
      mesh=mesh,
      scratch_types=[
          pltpu.VMEM((RPT, 128), jnp.int32),
          pltpu.VMEM((RPT, 128), jnp.float32),
          pltpu.VMEM((GPT, 128), jnp.int32),
          pltpu.VMEM((128,), jnp.float32),
          pltpu.SemaphoreType.DMA,
          pltpu.VMEM_SHARED((DPAD,), jnp.float32),
      ],
  )
  return f(obs_vec, zvi_pad, dst_pad, ones128, zeros1)


def _sc2_body(y_hbm, src_hbm, dst_hbm, zeros2_hbm, s_hbm,
              src_v, rows_a, rows_b, dst64_a, dst64_b,
              sem_a, sem_b, sem_da, sem_db, s_sp):
  c = jax.lax.axis_index("c")
  s = jax.lax.axis_index("s")
  wid = s * NC + c

  pltpu.sync_copy(zeros2_hbm.at[pl.ds(s * (DPAD // NS), DPAD // NS)],
                  s_sp.at[pl.ds(s * (DPAD // NS), DPAD // NS)])
  plsc.subcore_barrier()

  pltpu.sync_copy(src_hbm.at[pl.ds(wid * GPT, GPT)], src_v)
  ebase = wid * GPT * 128

  pltpu.async_copy(y_hbm.at[src_v.at[0, pl.ds(0, 64)]], rows_a, sem_a)
  pltpu.async_copy(dst_hbm.at[pl.ds(ebase, 64)], dst64_a, sem_da)

  @pl.loop(0, GPT)
  def _(jj):
    pltpu.make_async_copy(y_hbm.at[src_v.at[jj, pl.ds(0, 64)]],
                          rows_a, sem_a).wait()
    pltpu.async_copy(y_hbm.at[src_v.at[jj, pl.ds(64, 64)]], rows_b, sem_b)
    pltpu.async_copy(dst_hbm.at[pl.ds(ebase + jj * 128 + 64, 64)],
                     dst64_b, sem_db)
    pltpu.make_async_copy(dst_hbm.at[pl.ds(ebase, 64)], dst64_a, sem_da).wait()
    pltpu.sync_copy(rows_a, s_sp.at[dst64_a], add=True)

    pltpu.make_async_copy(y_hbm.at[src_v.at[jj, pl.ds(64, 64)]],
                          rows_b, sem_b).wait()

    @pl.when(jj < GPT - 1)
    def _():
      pltpu.async_copy(y_hbm.at[src_v.at[jj + 1, pl.ds(0, 64)]], rows_a, sem_a)
      pltpu.async_copy(dst_hbm.at[pl.ds(ebase + (jj + 1) * 128, 64)],
                       dst64_a, sem_da)

    pltpu.make_async_copy(dst_hbm.at[pl.ds(ebase, 64)], dst64_b, sem_db).wait()
    pltpu.sync_copy(rows_b, s_sp.at[dst64_b], add=True)

  plsc.subcore_barrier()
  pltpu.sync_copy(s_sp.at[pl.ds(s * (DPAD // NS), DPAD // NS)],
                  s_hbm.at[c].at[pl.ds(s * (DPAD // NS), DPAD // NS)])


def _sc2(y, src_pad, dst_pad, zeros2):
  mesh = plsc.VectorSubcoreMesh(core_axis_name="c", subcore_axis_name="s")
  f = pl.kernel(
      _sc2_body,
      out_type=jax.ShapeDtypeStruct((NC, DPAD, 128), jnp.float32),
      mesh=mesh,
      scratch_types=[
          pltpu.VMEM((GPT, 128), jnp.int32),
          pltpu.VMEM((64, 128), jnp.float32),
          pltpu.VMEM((64, 128), jnp.float32),
          pltpu.VMEM((64,), jnp.int32),
          pltpu.VMEM((64,), jnp.int32),
          pltpu.SemaphoreType.DMA,
          pltpu.SemaphoreType.DMA,
          pltpu.SemaphoreType.DMA,
          pltpu.SemaphoreType.DMA,
          pltpu.VMEM_SHARED((DPAD, 128), jnp.float32),
      ],
  )
  return f(y, src_pad, dst_pad, zeros2)


BN = 1000


def _tc1_body(f_ref, m_ref, d0_ref, d1_ref, we_ref, be_ref, wg_ref, y_ref):
  x = f_ref[...] * m_ref[...]
  h1 = jnp.maximum(
      jnp.dot(x, we_ref[...], preferred_element_type=jnp.float32) + be_ref[...],
      0.0)
  dinv = jax.lax.rsqrt(1.0 + d0_ref[...] + d1_ref[...])
  y_ref[...] = dinv * jnp.dot(h1, wg_ref[...],
                              preferred_element_type=jnp.float32)


def _tc1(feature, mask, d0, d1, W_enc, b_enc, W_gcn):
  grid = (N // BN,)
  return pl.pallas_call(
      _tc1_body,
      grid=grid,
      in_specs=[
          pl.BlockSpec((BN, 128), lambda i: (i, 0)),
          pl.BlockSpec((BN, 128), lambda i: (i, 0)),
          pl.BlockSpec((BN, 1), lambda i: (i, 0)),
          pl.BlockSpec((BN, 1), lambda i: (i, 0)),
          pl.BlockSpec((128, 128), lambda i: (0, 0)),
          pl.BlockSpec((1, 128), lambda i: (0, 0)),
          pl.BlockSpec((128, 128), lambda i: (0, 0)),
      ],
      out_specs=pl.BlockSpec((BN, 128), lambda i: (i, 0)),
      out_shape=jax.ShapeDtypeStruct((N, 128), jnp.float32),
  )(feature, mask, d0, d1, W_enc, b_enc, W_gcn)


def _tc2_body(s0_ref, s1_ref, y_ref, d0_ref, d1_ref, bg_ref,
              w1_ref, b1_ref, w2_ref, b2_ref, q_ref):
  dinv = jax.lax.rsqrt(1.0 + d0_ref[...] + d1_ref[...])
  agg = dinv * (s0_ref[...] + s1_ref[...] + y_ref[...])
  h2 = jnp.maximum(agg + bg_ref[...], 0.0)
  t = jnp.maximum(
      jnp.dot(h2, w1_ref[...], preferred_element_type=jnp.float32) + b1_ref[...],
      0.0)
  q_ref[...] = jnp.dot(t, w2_ref[...],
                       preferred_element_type=jnp.float32) + b2_ref[...]


def _tc2(s0, s1, y, d0, d1, b_gcn, W_q1, b_q1, W_q2p, b_q2p):
  grid = (N // BN,)
  return pl.pallas_call(
      _tc2_body,
      grid=grid,
      in_specs=[
          pl.BlockSpec((BN, 128), lambda i: (i, 0)),
          pl.BlockSpec((BN, 128), lambda i: (i, 0)),
          pl.BlockSpec((BN, 128), lambda i: (i, 0)),
          pl.BlockSpec((BN, 1), lambda i: (i, 0)),
          pl.BlockSpec((BN, 1), lambda i: (i, 0)),
          pl.BlockSpec((1, 128), lambda i: (0, 0)),
          pl.BlockSpec((128, 128), lambda i: (0, 0)),
          pl.BlockSpec((1, 128), lambda i: (0, 0)),
          pl.BlockSpec((128, 128), lambda i: (0, 0)),
          pl.BlockSpec((1, 128), lambda i: (0, 0)),
      ],
      out_specs=pl.BlockSpec((BN, 128), lambda i: (i, 0)),
      out_shape=jax.ShapeDtypeStruct((N, 128), jnp.float32),
  )(s0, s1, y, d0, d1, b_gcn, W_q1, b_q1, W_q2p, b_q2p)


@jax.jit
def kernel(obs_vec, zone_var_index, zone_mask, edge_index, W_enc, b_enc,
           W_gcn, b_gcn, W_q1, b_q1, W_q2, b_q2):
  zvi = zone_var_index.astype(jnp.int32)
  zvi_pad = jnp.concatenate([zvi, jnp.zeros((NP - N, F), jnp.int32)], axis=0)
  src = edge_index[0].astype(jnp.int32)
  dst = edge_index[1].astype(jnp.int32)
  src_pad = jnp.concatenate([src, jnp.zeros((EP - E,), jnp.int32)])
  dst_pad = jnp.concatenate([dst, jnp.full((EP - E,), N, jnp.int32)])
  src2d = src_pad.reshape(EP // 128, 128)
  dst2d = dst_pad.reshape(EP // 128, 128)
  ones128 = jnp.ones((128,), jnp.float32)
  zeros1 = jnp.zeros((DPAD,), jnp.float32)
  zeros2 = jnp.zeros((DPAD, 128), jnp.float32)

  feat_pad, deg = _sc1(obs_vec, zvi_pad, dst2d, ones128, zeros1)
  feature = feat_pad[:N]
  d0 = deg[0, :N].reshape(N, 1)
  d1 = deg[1, :N].reshape(N, 1)

  y = _tc1(feature, zone_mask, d0, d1, W_enc, b_enc.reshape(1, 128), W_gcn)

  s_part = _sc2(y, src2d, dst_pad, zeros2)

  W_q2p = jnp.pad(W_q2, ((0, 0), (0, 128 - A)))
  b_q2p = jnp.pad(b_q2, (0, 128 - A)).reshape(1, 128)
  qp = _tc2(s_part[0, :N], s_part[1, :N], y, d0, d1, b_gcn.reshape(1, 128),
            W_q1, b_q1.reshape(1, 128), W_q2p, b_q2p)
  return qp[:, :A]
